# Initial kernel scaffold; baseline (speedup 1.0000x reference)
import jax
import jax.numpy as jnp
from jax.experimental import pallas as pl


def kernel(edge_index, features, edge_weights, W, b):
    raise NotImplementedError("write your pallas kernel here")


# speedup vs baseline: 4.5347x; 4.5347x over previous
import functools

import jax
import jax.numpy as jnp
from jax import lax
from jax.experimental import pallas as pl
from jax.experimental.pallas import tpu as pltpu
from jax.experimental.pallas import tpu_sc as plsc

NC, NS, L = 2, 16, 16


@functools.lru_cache(maxsize=None)
def _make_sc_aggregate(N, E, D, npad):
    EW = E // (NC * NS)
    CH = 80
    assert E % (NC * NS) == 0 and EW % CH == 0 and D % L == 0
    nchunk = EW // CH
    rows_per_tile = npad // NS
    assert npad % NS == 0 and rows_per_tile % CH == 0

    mesh = plsc.VectorSubcoreMesh(core_axis_name="c", subcore_axis_name="s")

    @functools.partial(
        pl.kernel,
        mesh=mesh,
        out_type=jax.ShapeDtypeStruct((NC, npad, D), jnp.float32),
        scratch_types=[
            pltpu.VMEM((CH,), jnp.int32),
            pltpu.VMEM((CH,), jnp.int32),
            pltpu.VMEM((CH,), jnp.float32),
            pltpu.VMEM((CH, D), jnp.float32),
            pltpu.VMEM_SHARED((npad, D), jnp.float32),
            pltpu.SemaphoreType.DMA,
        ],
    )
    def agg(src_hbm, dst_hbm, ew_hbm, feat_hbm, out_hbm,
            src_v, dst_v, w_v, rows_v, acc_sh, sem):
        c = lax.axis_index("c")
        s = lax.axis_index("s")
        wid = s * NC + c
        base = wid * EW

        zvec = jnp.zeros((L,), jnp.float32)

        def zero_rows(j, _):
            for dd in range(D // L):
                rows_v[j, pl.ds(dd * L, L)] = zvec
            return 0

        lax.fori_loop(0, CH, zero_rows, 0)
        r0 = s * rows_per_tile

        def zero_acc(k, _):
            pltpu.sync_copy(rows_v, acc_sh.at[pl.ds(r0 + k * CH, CH)])
            return 0

        lax.fori_loop(0, rows_per_tile // CH, zero_acc, 0)
        plsc.subcore_barrier()

        def chunk_body(i, _):
            off = base + i * CH
            pltpu.sync_copy(src_hbm.at[pl.ds(off, CH)], src_v)
            pltpu.sync_copy(dst_hbm.at[pl.ds(off, CH)], dst_v)
            pltpu.sync_copy(ew_hbm.at[pl.ds(off, CH)], w_v)
            pltpu.async_copy(feat_hbm.at[src_v], rows_v, sem).wait()

            def scale_group(g, _):
                w16 = w_v[pl.ds(g * L, L)]
                for j in range(L):
                    w = w16[j]
                    r = g * L + j
                    for dd in range(D // L):
                        sl = pl.ds(dd * L, L)
                        rows_v[r, sl] = rows_v[r, sl] * w
                return 0

            lax.fori_loop(0, CH // L, scale_group, 0)
            pltpu.sync_copy(rows_v, acc_sh.at[dst_v], add=True)
            return 0

        lax.fori_loop(0, nchunk, chunk_body, 0)

        plsc.subcore_barrier()
        pltpu.sync_copy(acc_sh.at[pl.ds(r0, rows_per_tile)],
                        out_hbm.at[c, pl.ds(r0, rows_per_tile)])

    return agg


def _tc_finish_body(agg_ref, w_ref, b_ref, o_ref):
    a = agg_ref[0] + agg_ref[1]
    z = jnp.dot(a, w_ref[...], preferred_element_type=jnp.float32) + b_ref[...]
    m = jnp.max(z, axis=1, keepdims=True)
    e = jnp.exp(z - m)
    ssum = jnp.sum(e, axis=1, keepdims=True)
    o_ref[...] = z - m - jnp.log(ssum)


@functools.lru_cache(maxsize=None)
def _make_tc_finish(npad, D, block_rows=1024):
    assert npad % block_rows == 0
    grid = (npad // block_rows,)
    return pl.pallas_call(
        _tc_finish_body,
        grid=grid,
        in_specs=[
            pl.BlockSpec((NC, block_rows, D), lambda i: (0, i, 0)),
            pl.BlockSpec((D, D), lambda i: (0, 0)),
            pl.BlockSpec((1, D), lambda i: (0, 0)),
        ],
        out_specs=pl.BlockSpec((block_rows, D), lambda i: (i, 0)),
        out_shape=jax.ShapeDtypeStruct((npad, D), jnp.float32),
    )


def kernel(edge_index, features, edge_weights, W, b):
    N, D = features.shape
    E = edge_index.shape[1]
    npad = -(-N // 5120) * 5120
    src = edge_index[0]
    dst = edge_index[1]
    agg = _make_sc_aggregate(N, E, D, npad)(src, dst, edge_weights, features)
    out = _make_tc_finish(npad, D)(agg, W, b.reshape(1, D))
    return out[:N]
